# baseline (speedup 1.0000x reference)
import jax
import jax.numpy as jnp
from jax import lax
from jax.experimental import pallas as pl
from jax.experimental.pallas import tpu as pltpu
from jax.experimental.pallas import tpu_sc as plsc

VOCAB = 8192
B, T = 16, 512
N_TOK = B * T

_info = plsc.get_sparse_core_info()
NC, NS = _info.num_cores, _info.num_subcores
NW = NC * NS
TOK_PER_W = N_TOK // NW
HD = VOCAB // 2
ROWS_PER_W = 2 * TOK_PER_W
CH = 8
NCHUNK = ROWS_PER_W // CH
L = 16


def _gather_body(idx_hbm, table_hbm, out_hbm, idx_v, idx2_v, buf0, buf1,
                 g0, g1, w0, w1):
    wid = lax.axis_index("s") * NC + lax.axis_index("c")
    base = wid * TOK_PER_W
    pltpu.sync_copy(idx_hbm.at[pl.ds(base, TOK_PER_W)], idx_v)

    lanes = lax.iota(jnp.int32, L)
    for m in range(ROWS_PER_W // L):
        k = lanes + m * L
        src = lax.shift_right_logical(k, 1)
        v = plsc.load_gather(idx_v, [src])
        idx2_v[pl.ds(m * L, L)] = v * 2 + lax.bitwise_and(k, 1)

    bufs = (buf0, buf1)
    gsems = (g0, g1)
    wsems = (w0, w1)
    obase = wid * ROWS_PER_W

    def start_gather(u, b):
        pltpu.make_async_copy(
            table_hbm.at[idx2_v.at[pl.ds(u * CH, CH)]], bufs[b], gsems[b]
        ).start()

    def wait_gather(b):
        pltpu.make_async_copy(
            table_hbm.at[idx2_v.at[pl.ds(0, CH)]], bufs[b], gsems[b]
        ).wait()

    def start_write(u, b):
        pltpu.make_async_copy(
            bufs[b], out_hbm.at[pl.ds(obase + u * CH, CH)], wsems[b]
        ).start()

    def wait_write(b):
        pltpu.make_async_copy(
            bufs[b], out_hbm.at[pl.ds(obase, CH)], wsems[b]
        ).wait()

    def step(u, carry):
        pltpu.async_copy(
            table_hbm.at[idx2_v.at[pl.ds(u * CH, CH)]], buf0, g0
        ).wait()
        pltpu.sync_copy(buf0, out_hbm.at[pl.ds(obase + u * CH, CH)])
        return carry

    lax.fori_loop(0, NCHUNK, step, 0)


@jax.jit
def _gather(idx_flat, table2):
    mesh = plsc.VectorSubcoreMesh(core_axis_name="c", subcore_axis_name="s")
    return pl.kernel(
        _gather_body,
        out_type=jax.ShapeDtypeStruct((2 * N_TOK, HD), jnp.float32),
        mesh=mesh,
        compiler_params=pltpu.CompilerParams(needs_layout_passes=False),
        scratch_types=[
            pltpu.VMEM((TOK_PER_W,), jnp.int32),
            pltpu.VMEM((ROWS_PER_W,), jnp.int32),
            pltpu.VMEM((CH, HD), jnp.float32),
            pltpu.VMEM((CH, HD), jnp.float32),
            pltpu.SemaphoreType.DMA,
            pltpu.SemaphoreType.DMA,
            pltpu.SemaphoreType.DMA,
            pltpu.SemaphoreType.DMA,
        ],
    )(idx_flat, table2)


def kernel(idx, table):
    idx_flat = idx.reshape(N_TOK).astype(jnp.int32)
    out = _gather(idx_flat, table.reshape(2 * VOCAB, HD))
    return out.reshape(B, T, VOCAB)


# speedup vs baseline: 3.5712x; 3.5712x over previous
import jax
import jax.numpy as jnp
from jax import lax
from jax.experimental import pallas as pl
from jax.experimental.pallas import tpu as pltpu
from jax.experimental.pallas import tpu_sc as plsc

VOCAB = 8192
B, T = 16, 512
N_TOK = B * T

_info = plsc.get_sparse_core_info()
NC, NS = _info.num_cores, _info.num_subcores
NW = NC * NS
TOK_PER_W = N_TOK // NW
CH = 8
NCHUNK = TOK_PER_W // CH


def _gather_body(idx_hbm, table_hbm, out_hbm, idx_v, buf0, g0):
    cid = lax.axis_index("c")
    sid = lax.axis_index("s")
    wid = sid * NC + cid
    base = wid * TOK_PER_W
    pltpu.sync_copy(idx_hbm.at[pl.ds(base, TOK_PER_W)], idx_v)

    def step(u, carry):
        pltpu.async_copy(
            table_hbm.at[idx_v.at[pl.ds(u * CH, CH)]], buf0, g0
        ).wait()
        pltpu.sync_copy(buf0, out_hbm.at[pl.ds(base + u * CH, CH)])
        return carry

    lax.fori_loop(0, NCHUNK, step, 0)


@jax.jit
def _gather(idx_flat, table):
    mesh = plsc.VectorSubcoreMesh(core_axis_name="c", subcore_axis_name="s")
    return pl.kernel(
        _gather_body,
        out_type=jax.ShapeDtypeStruct((N_TOK, VOCAB), jnp.float32),
        mesh=mesh,
        compiler_params=pltpu.CompilerParams(needs_layout_passes=False),
        scratch_types=[
            pltpu.VMEM((TOK_PER_W,), jnp.int32),
            pltpu.VMEM((CH, VOCAB), jnp.float32),
            pltpu.SemaphoreType.DMA,
        ],
    )(idx_flat, table)


def kernel(idx, table):
    idx_flat = idx.reshape(N_TOK).astype(jnp.int32)
    out = _gather(idx_flat, table)
    return out.reshape(B, T, VOCAB)
